# baseline (speedup 1.0000x reference)
import functools

import jax
import jax.numpy as jnp
from jax import lax
from jax.experimental import pallas as pl
from jax.experimental.pallas import tpu as pltpu
from jax.experimental.pallas import tpu_sc as plsc

N = 10000
E = 320000
F_IN = 128
F_HID = 128
F_OUT = 64

NC = 2
NS = 16
NW = NC * NS
E_PW = E // NW
B = 80
NCHUNK = E_PW // B
N_PAD = 10240
R_PS = N_PAD // NS

_mesh = plsc.VectorSubcoreMesh(core_axis_name="c", subcore_axis_name="s")


def _deg_body(edge_hbm, out_hbm, dst_v, hist_v):
    c = lax.axis_index("c")
    s = lax.axis_index("s")
    wid = c * NS + s
    pltpu.sync_copy(edge_hbm.at[1, pl.ds(wid * E_PW, E_PW)], dst_v)

    zeros16 = jnp.zeros((16,), jnp.float32)
    ones16 = jnp.ones((16,), jnp.float32)

    def zero_body(i, _):
        hist_v[pl.ds(i * 16, 16)] = zeros16
        return 0

    lax.fori_loop(0, N // 16, zero_body, 0)

    def acc_body(i, _):
        idx = dst_v[pl.ds(i * 16, 16)]
        plsc.addupdate_scatter(hist_v, [idx], ones16)
        return 0

    lax.fori_loop(0, E_PW // 16, acc_body, 0)
    pltpu.sync_copy(hist_v, out_hbm.at[wid])


@functools.partial(
    pl.kernel,
    out_type=jax.ShapeDtypeStruct((NW, N), jnp.float32),
    mesh=_mesh,
    scratch_types=[
        pltpu.VMEM((E_PW,), jnp.int32),
        pltpu.VMEM((N,), jnp.float32),
    ],
    compiler_params=pltpu.CompilerParams(
        needs_layout_passes=False, use_tc_tiling_on_sc=False),
)
def _deg_kernel(edge_hbm, out_hbm, dst_v, hist_v):
    _deg_body(edge_hbm, out_hbm, dst_v, hist_v)


def _seg_body(D, y_hbm, edge_hbm, dstr_hbm, out_hbm,
              src_v, dst_v, g0, g1, acc_sh, gsem0, gsem1, ssem, dsem):
    c = lax.axis_index("c")
    s = lax.axis_index("s")
    wid = c * NS + s
    base = wid * E_PW

    dst_copy = pltpu.make_async_copy(dstr_hbm.at[wid], dst_v, dsem)
    dst_copy.start()
    pltpu.sync_copy(edge_hbm.at[0, pl.ds(base, E_PW)], src_v)

    zeros16 = jnp.zeros((16,), jnp.float32)
    lanes = D // 16

    def zero_body(i, _):
        g0[i // lanes, pl.ds((i % lanes) * 16, 16)] = zeros16
        return 0

    lax.fori_loop(0, B * lanes, zero_body, 0)
    for k in range(R_PS // B):
        pltpu.sync_copy(g0, acc_sh.at[pl.ds(s * R_PS + k * B, B)])
    dst_copy.wait()
    plsc.subcore_barrier()

    def gather_start(j, gbuf, sem):
        pltpu.make_async_copy(
            y_hbm.at[src_v.at[pl.ds(j * B, B)]], gbuf, sem).start()

    def gather_wait(j, gbuf, sem):
        pltpu.make_async_copy(
            y_hbm.at[src_v.at[pl.ds(j * B, B)]], gbuf, sem).wait()

    def scat(j, gbuf):
        copy = pltpu.make_async_copy(gbuf, acc_sh.at[dst_v.at[j]], ssem)
        copy.start(add=True)
        copy.wait()

    gather_start(0, g0, gsem0)

    def loop_body(jj, _):
        j0 = jj * 2
        gather_start(j0 + 1, g1, gsem1)
        gather_wait(j0, g0, gsem0)
        scat(j0, g0)
        gather_start(j0 + 2, g0, gsem0)
        gather_wait(j0 + 1, g1, gsem1)
        scat(j0 + 1, g1)
        return 0

    lax.fori_loop(0, (NCHUNK - 1) // 2, loop_body, 0)
    gather_wait(NCHUNK - 1, g0, gsem0)
    scat(NCHUNK - 1, g0)

    plsc.subcore_barrier()
    pltpu.sync_copy(acc_sh.at[pl.ds(s * R_PS, R_PS)],
                    out_hbm.at[c, pl.ds(s * R_PS, R_PS)])


def _make_seg_kernel(D):
    @functools.partial(
        pl.kernel,
        out_type=jax.ShapeDtypeStruct((NC, N_PAD, D), jnp.float32),
        mesh=_mesh,
        scratch_types=[
            pltpu.VMEM((E_PW,), jnp.int32),
            pltpu.VMEM((NCHUNK, B), jnp.int32),
            pltpu.VMEM((B, D), jnp.float32),
            pltpu.VMEM((B, D), jnp.float32),
            pltpu.VMEM_SHARED((N_PAD, D), jnp.float32),
            pltpu.SemaphoreType.DMA,
            pltpu.SemaphoreType.DMA,
            pltpu.SemaphoreType.DMA,
            pltpu.SemaphoreType.DMA,
        ],
        compiler_params=pltpu.CompilerParams(use_tc_tiling_on_sc=False),
    )
    def seg_kernel(y_hbm, edge_hbm, dstr_hbm, out_hbm, *rest):
        _seg_body(D, y_hbm, edge_hbm, dstr_hbm, out_hbm, *rest)

    return seg_kernel


_seg128 = _make_seg_kernel(F_HID)
_seg64 = _make_seg_kernel(F_OUT)


_GRID = 5
_RB = N // _GRID
_RBP = _RB // 2


def _dinv_from_parts(parts_blk):
    deg = 1.0 + jnp.sum(parts_blk, axis=1)
    return lax.rsqrt(deg)


def _pack64(x):
    x3 = x.reshape(-1, 2, F_OUT)
    return jnp.concatenate([x3[:, 0, :], x3[:, 1, :]], axis=-1)


def _unpack64(p):
    a = p[:, :F_OUT][:, None, :]
    b = p[:, F_OUT:][:, None, :]
    return jnp.concatenate([a, b], axis=1).reshape(-1, F_OUT)


def _tc1a_body(x_ref, w1_ref, xw_ref):
    xw_ref[...] = jnp.dot(x_ref[...], w1_ref[...],
                          preferred_element_type=jnp.float32)


def _tc1a(x, W1):
    return pl.pallas_call(
        _tc1a_body,
        grid=(_GRID,),
        in_specs=[
            pl.BlockSpec((_RB, F_IN), lambda i: (i, 0)),
            pl.BlockSpec((F_IN, F_HID), lambda i: (0, 0)),
        ],
        out_specs=pl.BlockSpec((_RB, F_HID), lambda i: (i, 0)),
        out_shape=jax.ShapeDtypeStruct((N, F_HID), jnp.float32),
    )(x, W1)


def _tc1b_body(parts_ref, xw_ref, y1_ref):
    dinv = _dinv_from_parts(parts_ref[...])
    y1_ref[...] = xw_ref[...] * dinv[:, None]


def _tc1b(parts, xw):
    return pl.pallas_call(
        _tc1b_body,
        grid=(_GRID,),
        in_specs=[
            pl.BlockSpec((_RB, NW), lambda i: (i, 0)),
            pl.BlockSpec((_RB, F_HID), lambda i: (i, 0)),
        ],
        out_specs=pl.BlockSpec((_RB, F_HID), lambda i: (i, 0)),
        out_shape=jax.ShapeDtypeStruct((N, F_HID), jnp.float32),
    )(parts, xw)


def _tc2_body(parts_ref, s1_ref, y1_ref, b1_ref, w2_ref, y2_ref):
    dinv = _dinv_from_parts(parts_ref[...])
    ssum = s1_ref[0] + s1_ref[1]
    h1 = dinv[:, None] * (ssum + y1_ref[...]) + b1_ref[...]
    h1 = jnp.maximum(h1, 0.0)
    xw = jnp.dot(h1, w2_ref[...], preferred_element_type=jnp.float32)
    y2_ref[...] = _pack64(xw * dinv[:, None])


def _tc2(parts, s1, y1, b1, W2):
    return pl.pallas_call(
        _tc2_body,
        grid=(_GRID,),
        in_specs=[
            pl.BlockSpec((_RB, NW), lambda i: (i, 0)),
            pl.BlockSpec((NC, _RB, F_HID), lambda i: (0, i, 0)),
            pl.BlockSpec((_RB, F_HID), lambda i: (i, 0)),
            pl.BlockSpec((1, F_HID), lambda i: (0, 0)),
            pl.BlockSpec((F_HID, F_OUT), lambda i: (0, 0)),
        ],
        out_specs=pl.BlockSpec((_RBP, F_HID), lambda i: (i, 0)),
        out_shape=jax.ShapeDtypeStruct((N // 2, F_HID), jnp.float32),
    )(parts, s1, y1, b1, W2)


def _tc3_body(partsp_ref, s2p_ref, y2p_ref, b2p_ref, out_ref):
    pp = partsp_ref[...]
    dinv_e = lax.rsqrt(1.0 + jnp.sum(pp[:, :NW], axis=1))[:, None]
    dinv_o = lax.rsqrt(1.0 + jnp.sum(pp[:, NW:], axis=1))[:, None]
    dinvp = jnp.concatenate(
        [jnp.broadcast_to(dinv_e, (_RBP, F_OUT)),
         jnp.broadcast_to(dinv_o, (_RBP, F_OUT))], axis=-1)
    s2p = s2p_ref[...]
    h2p = dinvp * (s2p[0] + s2p[1] + y2p_ref[...]) + b2p_ref[...]
    ex = jnp.exp(h2p)
    r = lax.broadcasted_iota(jnp.int32, (F_HID, F_HID), 0) // F_OUT
    c = lax.broadcasted_iota(jnp.int32, (F_HID, F_HID), 1) // F_OUT
    mask = (r == c).astype(jnp.float32)
    gs = jnp.dot(ex, mask, preferred_element_type=jnp.float32)
    out_ref[...] = _unpack64(h2p - jnp.log(gs))


def _tc3(partsp, s2p, y2p, b2p):
    return pl.pallas_call(
        _tc3_body,
        grid=(_GRID,),
        in_specs=[
            pl.BlockSpec((_RBP, 2 * NW), lambda i: (i, 0)),
            pl.BlockSpec((NC, _RBP, F_HID), lambda i: (0, i, 0)),
            pl.BlockSpec((_RBP, F_HID), lambda i: (i, 0)),
            pl.BlockSpec((1, F_HID), lambda i: (0, 0)),
        ],
        out_specs=pl.BlockSpec((_RB, F_OUT), lambda i: (i, 0)),
        out_shape=jax.ShapeDtypeStruct((N, F_OUT), jnp.float32),
    )(partsp, s2p, y2p, b2p)


def kernel(x, edge_index, W1, b1, W2, b2):
    e32 = edge_index.astype(jnp.int32)

    dstr = e32[1].reshape(NW, NCHUNK, B)
    xw1 = _tc1a(x, W1)
    deg_parts = _deg_kernel(e32).T
    y1 = _tc1b(deg_parts, xw1)
    s1 = _seg128(y1, e32, dstr)
    y2p = _tc2(deg_parts, s1, y1, b1.reshape(1, F_HID), W2)
    s2 = _seg64(y2p.reshape(N, F_OUT), e32, dstr)
    s2p = s2.reshape(NC, N_PAD // 2, F_HID)
    partsp = deg_parts.reshape(N // 2, 2 * NW)
    b2p = jnp.concatenate([b2, b2]).reshape(1, F_HID)
    return _tc3(partsp, s2p, y2p, b2p)


# speedup vs baseline: 1.0439x; 1.0439x over previous
import functools

import jax
import jax.numpy as jnp
from jax import lax
from jax.experimental import pallas as pl
from jax.experimental.pallas import tpu as pltpu
from jax.experimental.pallas import tpu_sc as plsc

N = 10000
E = 320000
F_IN = 128
F_HID = 128
F_OUT = 64

NC = 2
NS = 16
NW = NC * NS
E_PW = E // NW
B = 80
NCHUNK = E_PW // B
N_PAD = 10240
R_PS = N_PAD // NS

_mesh = plsc.VectorSubcoreMesh(core_axis_name="c", subcore_axis_name="s")


def _deg_body(edge_hbm, out_hbm, dst_v, hist_v):
    c = lax.axis_index("c")
    s = lax.axis_index("s")
    wid = c * NS + s
    pltpu.sync_copy(edge_hbm.at[1, pl.ds(wid * E_PW, E_PW)], dst_v)

    zeros16 = jnp.zeros((16,), jnp.float32)
    ones16 = jnp.ones((16,), jnp.float32)

    def zero_body(i, _):
        hist_v[pl.ds(i * 16, 16)] = zeros16
        return 0

    lax.fori_loop(0, N // 16, zero_body, 0)

    def acc_body(i, _):
        idx = dst_v[pl.ds(i * 16, 16)]
        plsc.addupdate_scatter(hist_v, [idx], ones16)
        return 0

    lax.fori_loop(0, E_PW // 16, acc_body, 0)
    pltpu.sync_copy(hist_v, out_hbm.at[wid])


@functools.partial(
    pl.kernel,
    out_type=jax.ShapeDtypeStruct((NW, N), jnp.float32),
    mesh=_mesh,
    scratch_types=[
        pltpu.VMEM((E_PW,), jnp.int32),
        pltpu.VMEM((N,), jnp.float32),
    ],
    compiler_params=pltpu.CompilerParams(
        needs_layout_passes=False, use_tc_tiling_on_sc=False),
)
def _deg_kernel(edge_hbm, out_hbm, dst_v, hist_v):
    _deg_body(edge_hbm, out_hbm, dst_v, hist_v)


def _seg_body(D, y_hbm, edge_hbm, out_hbm,
              src_v, dst_v, g0, g1, acc_sh, gsem0, gsem1, ssem, dsem):
    c = lax.axis_index("c")
    s = lax.axis_index("s")
    wid = c * NS + s
    base = wid * E_PW

    def dst_load_start(j, _):
        pltpu.make_async_copy(
            edge_hbm.at[1, pl.ds(base + j * B, B)], dst_v.at[j], dsem).start()
        return 0

    def dst_load_wait(j, _):
        pltpu.make_async_copy(
            edge_hbm.at[1, pl.ds(base + j * B, B)], dst_v.at[j], dsem).wait()
        return 0

    lax.fori_loop(0, NCHUNK, dst_load_start, 0)
    pltpu.sync_copy(edge_hbm.at[0, pl.ds(base, E_PW)], src_v)

    def gather_start(j, gbuf, sem):
        pltpu.make_async_copy(
            y_hbm.at[src_v.at[pl.ds(j * B, B)]], gbuf, sem).start()

    def gather_wait(j, gbuf, sem):
        pltpu.make_async_copy(
            y_hbm.at[src_v.at[pl.ds(j * B, B)]], gbuf, sem).wait()

    def scat(j, gbuf):
        copy = pltpu.make_async_copy(gbuf, acc_sh.at[dst_v.at[j]], ssem)
        copy.start(add=True)
        copy.wait()

    gather_start(0, g1, gsem1)

    zeros16 = jnp.zeros((16,), jnp.float32)
    lanes = D // 16

    def zero_body(i, _):
        g0[i // lanes, pl.ds((i % lanes) * 16, 16)] = zeros16
        return 0

    lax.fori_loop(0, B * lanes, zero_body, 0)
    for k in range(R_PS // B):
        pltpu.sync_copy(g0, acc_sh.at[pl.ds(s * R_PS + k * B, B)])
    lax.fori_loop(0, NCHUNK, dst_load_wait, 0)
    plsc.subcore_barrier()

    def loop_body(jj, _):
        j0 = jj * 2
        gather_start(j0 + 1, g0, gsem0)
        gather_wait(j0, g1, gsem1)
        scat(j0, g1)
        gather_start(j0 + 2, g1, gsem1)
        gather_wait(j0 + 1, g0, gsem0)
        scat(j0 + 1, g0)
        return 0

    lax.fori_loop(0, (NCHUNK - 1) // 2, loop_body, 0)
    gather_wait(NCHUNK - 1, g1, gsem1)
    scat(NCHUNK - 1, g1)

    plsc.subcore_barrier()
    pltpu.sync_copy(acc_sh.at[pl.ds(s * R_PS, R_PS)],
                    out_hbm.at[c, pl.ds(s * R_PS, R_PS)])


def _make_seg_kernel(D):
    @functools.partial(
        pl.kernel,
        out_type=jax.ShapeDtypeStruct((NC, N_PAD, D), jnp.float32),
        mesh=_mesh,
        scratch_types=[
            pltpu.VMEM((E_PW,), jnp.int32),
            pltpu.VMEM((NCHUNK, B), jnp.int32),
            pltpu.VMEM((B, D), jnp.float32),
            pltpu.VMEM((B, D), jnp.float32),
            pltpu.VMEM_SHARED((N_PAD, D), jnp.float32),
            pltpu.SemaphoreType.DMA,
            pltpu.SemaphoreType.DMA,
            pltpu.SemaphoreType.DMA,
            pltpu.SemaphoreType.DMA,
        ],
        compiler_params=pltpu.CompilerParams(use_tc_tiling_on_sc=False),
    )
    def seg_kernel(y_hbm, edge_hbm, out_hbm, *rest):
        _seg_body(D, y_hbm, edge_hbm, out_hbm, *rest)

    return seg_kernel


_seg128 = _make_seg_kernel(F_HID)
_seg64 = _make_seg_kernel(F_OUT)


_GRID = 5
_RB = N // _GRID
_RBP = _RB // 2


def _dinv_from_parts(parts_blk):
    deg = 1.0 + jnp.sum(parts_blk, axis=1)
    return lax.rsqrt(deg)


def _pack64(x):
    x3 = x.reshape(-1, 2, F_OUT)
    return jnp.concatenate([x3[:, 0, :], x3[:, 1, :]], axis=-1)


def _unpack64(p):
    a = p[:, :F_OUT][:, None, :]
    b = p[:, F_OUT:][:, None, :]
    return jnp.concatenate([a, b], axis=1).reshape(-1, F_OUT)


def _tc1a_body(x_ref, w1_ref, xw_ref):
    xw_ref[...] = jnp.dot(x_ref[...], w1_ref[...],
                          preferred_element_type=jnp.float32)


def _tc1a(x, W1):
    return pl.pallas_call(
        _tc1a_body,
        grid=(_GRID,),
        in_specs=[
            pl.BlockSpec((_RB, F_IN), lambda i: (i, 0)),
            pl.BlockSpec((F_IN, F_HID), lambda i: (0, 0)),
        ],
        out_specs=pl.BlockSpec((_RB, F_HID), lambda i: (i, 0)),
        out_shape=jax.ShapeDtypeStruct((N, F_HID), jnp.float32),
    )(x, W1)


def _tc1b_body(parts_ref, xw_ref, y1_ref):
    dinv = _dinv_from_parts(parts_ref[...])
    y1_ref[...] = xw_ref[...] * dinv[:, None]


def _tc1b(parts, xw):
    return pl.pallas_call(
        _tc1b_body,
        grid=(_GRID,),
        in_specs=[
            pl.BlockSpec((_RB, NW), lambda i: (i, 0)),
            pl.BlockSpec((_RB, F_HID), lambda i: (i, 0)),
        ],
        out_specs=pl.BlockSpec((_RB, F_HID), lambda i: (i, 0)),
        out_shape=jax.ShapeDtypeStruct((N, F_HID), jnp.float32),
    )(parts, xw)


def _tc2_body(parts_ref, s1_ref, y1_ref, b1_ref, w2_ref, y2_ref):
    dinv = _dinv_from_parts(parts_ref[...])
    ssum = s1_ref[0] + s1_ref[1]
    h1 = dinv[:, None] * (ssum + y1_ref[...]) + b1_ref[...]
    h1 = jnp.maximum(h1, 0.0)
    xw = jnp.dot(h1, w2_ref[...], preferred_element_type=jnp.float32)
    y2_ref[...] = _pack64(xw * dinv[:, None])


def _tc2(parts, s1, y1, b1, W2):
    return pl.pallas_call(
        _tc2_body,
        grid=(_GRID,),
        in_specs=[
            pl.BlockSpec((_RB, NW), lambda i: (i, 0)),
            pl.BlockSpec((NC, _RB, F_HID), lambda i: (0, i, 0)),
            pl.BlockSpec((_RB, F_HID), lambda i: (i, 0)),
            pl.BlockSpec((1, F_HID), lambda i: (0, 0)),
            pl.BlockSpec((F_HID, F_OUT), lambda i: (0, 0)),
        ],
        out_specs=pl.BlockSpec((_RBP, F_HID), lambda i: (i, 0)),
        out_shape=jax.ShapeDtypeStruct((N // 2, F_HID), jnp.float32),
    )(parts, s1, y1, b1, W2)


def _tc3_body(partsp_ref, s2p_ref, y2p_ref, b2p_ref, out_ref):
    pp = partsp_ref[...]
    dinv_e = lax.rsqrt(1.0 + jnp.sum(pp[:, :NW], axis=1))[:, None]
    dinv_o = lax.rsqrt(1.0 + jnp.sum(pp[:, NW:], axis=1))[:, None]
    dinvp = jnp.concatenate(
        [jnp.broadcast_to(dinv_e, (_RBP, F_OUT)),
         jnp.broadcast_to(dinv_o, (_RBP, F_OUT))], axis=-1)
    s2p = s2p_ref[...]
    h2p = dinvp * (s2p[0] + s2p[1] + y2p_ref[...]) + b2p_ref[...]
    ex = jnp.exp(h2p)
    r = lax.broadcasted_iota(jnp.int32, (F_HID, F_HID), 0) // F_OUT
    c = lax.broadcasted_iota(jnp.int32, (F_HID, F_HID), 1) // F_OUT
    mask = (r == c).astype(jnp.float32)
    gs = jnp.dot(ex, mask, preferred_element_type=jnp.float32)
    out_ref[...] = _unpack64(h2p - jnp.log(gs))


def _tc3(partsp, s2p, y2p, b2p):
    return pl.pallas_call(
        _tc3_body,
        grid=(_GRID,),
        in_specs=[
            pl.BlockSpec((_RBP, 2 * NW), lambda i: (i, 0)),
            pl.BlockSpec((NC, _RBP, F_HID), lambda i: (0, i, 0)),
            pl.BlockSpec((_RBP, F_HID), lambda i: (i, 0)),
            pl.BlockSpec((1, F_HID), lambda i: (0, 0)),
        ],
        out_specs=pl.BlockSpec((_RB, F_OUT), lambda i: (i, 0)),
        out_shape=jax.ShapeDtypeStruct((N, F_OUT), jnp.float32),
    )(partsp, s2p, y2p, b2p)


def kernel(x, edge_index, W1, b1, W2, b2):
    e32 = edge_index.astype(jnp.int32)

    xw1 = _tc1a(x, W1)
    deg_parts = _deg_kernel(e32).T
    y1 = _tc1b(deg_parts, xw1)
    s1 = _seg128(y1, e32)
    y2p = _tc2(deg_parts, s1, y1, b1.reshape(1, F_HID), W2)
    s2 = _seg64(y2p.reshape(N, F_OUT), e32)
    s2p = s2.reshape(NC, N_PAD // 2, F_HID)
    partsp = deg_parts.reshape(N // 2, 2 * NW)
    b2p = jnp.concatenate([b2, b2]).reshape(1, F_HID)
    return _tc3(partsp, s2p, y2p, b2p)
